# Initial kernel scaffold; baseline (speedup 1.0000x reference)
import jax
import jax.numpy as jnp
from jax.experimental import pallas as pl


def kernel(x, edge_index, batch, conv1_Wl, conv1_bl, conv1_Wr, conv2_Wl, conv2_bl, conv2_Wr, conv3_Wl, conv3_bl, conv3_Wr, conv4_Wl, conv4_bl, conv4_Wr, lin1_W, lin1_b, lin2_W, lin2_b, lin3_W, lin3_b, lin4_W, lin4_b):
    raise NotImplementedError("write your pallas kernel here")


# speedup vs baseline: 2.7998x; 2.7998x over previous
import functools

import jax
import jax.numpy as jnp
from jax import lax
from jax.experimental import pallas as pl
from jax.experimental.pallas import tpu as pltpu
from jax.experimental.pallas import tpu_sc as plsc

N = 10000
E = 320000
D = 128
G = 128

NC = 2
NS = 16
NW = NC * NS

NP = 10240
RPS = NP // NS

C = 128
EPAD = 327680
EW = EPAD // NW
NCH = EW // C
NDR = RPS // C

BN = 2000

_F32 = jnp.float32


def _sc_segsum(feat, src3, dst3, zeros):
    mesh = plsc.VectorSubcoreMesh(core_axis_name="c", subcore_axis_name="s")

    @functools.partial(
        pl.kernel,
        out_type=jax.ShapeDtypeStruct((NC * NP, D), _F32),
        mesh=mesh,
        scratch_types=[
            pltpu.VMEM((NCH, C), jnp.int32),
            pltpu.VMEM((NCH, C), jnp.int32),
            pltpu.VMEM((C, D), _F32),
            pltpu.VMEM_SHARED((NP, D), _F32),
            pltpu.SemaphoreType.DMA,
        ],
    )
    def k(feat_h, src_h, dst_h, zero_h, out_h, isrc, idst, rows, shared, sem):
        c = lax.axis_index("c")
        s = lax.axis_index("s")
        wid = s * NC + c
        pltpu.sync_copy(zero_h, rows)
        for t in range(NDR):
            pltpu.sync_copy(rows, shared.at[pl.ds(s * RPS + t * C, C)])
        pltpu.sync_copy(src_h.at[wid], isrc)
        pltpu.sync_copy(dst_h.at[wid], idst)
        plsc.subcore_barrier()

        def body(j, carry):
            pltpu.async_copy(feat_h.at[isrc.at[j]], rows, sem).wait()
            pltpu.sync_copy(rows, shared.at[idst.at[j]], add=True)
            return carry

        lax.fori_loop(0, NCH, body, 0)
        plsc.subcore_barrier()
        for t in range(NDR):
            pltpu.sync_copy(shared.at[pl.ds(s * RPS + t * C, C)], rows)
            pltpu.sync_copy(rows, out_h.at[pl.ds(c * NP + s * RPS + t * C, C)])

    return k(feat, src3, dst3, zeros)


def _sc_segcnt(dst3, zeros, ones):
    mesh = plsc.VectorSubcoreMesh(core_axis_name="c", subcore_axis_name="s")

    @functools.partial(
        pl.kernel,
        out_type=jax.ShapeDtypeStruct((NC * NP, D), _F32),
        mesh=mesh,
        scratch_types=[
            pltpu.VMEM((NCH, C), jnp.int32),
            pltpu.VMEM((C, D), _F32),
            pltpu.VMEM_SHARED((NP, D), _F32),
        ],
    )
    def k(dst_h, zero_h, ones_h, out_h, idst, rows, shared):
        c = lax.axis_index("c")
        s = lax.axis_index("s")
        wid = s * NC + c
        pltpu.sync_copy(zero_h, rows)
        for t in range(NDR):
            pltpu.sync_copy(rows, shared.at[pl.ds(s * RPS + t * C, C)])
        pltpu.sync_copy(ones_h, rows)
        pltpu.sync_copy(dst_h.at[wid], idst)
        plsc.subcore_barrier()

        def body(j, carry):
            pltpu.sync_copy(rows, shared.at[idst.at[j]], add=True)
            return carry

        lax.fori_loop(0, NCH, body, 0)
        plsc.subcore_barrier()
        for t in range(NDR):
            pltpu.sync_copy(shared.at[pl.ds(s * RPS + t * C, C)], rows)
            pltpu.sync_copy(rows, out_h.at[pl.ds(c * NP + s * RPS + t * C, C)])

    return k(dst3, zeros, ones)


def _dgT(a, b):
    return lax.dot_general(a, b, (((1,), (1,)), ((), ())),
                           preferred_element_type=_F32,
                           precision=lax.Precision.HIGHEST)


def _tc_layer(h, m0, m1, c0, c1, Wl, bl, Wr, Wn):
    din = h.shape[1]
    dout = Wl.shape[0]
    have_wr = Wr is not None
    have_wn = Wn is not None

    def body(*args):
        h_r, m0_r, m1_r, c0_r, c1_r, wl_r, bl_r = args[:7]
        k = 7
        wr_r = wn_r = None
        if have_wr:
            wr_r = args[k]; k += 1
        if have_wn:
            wn_r = args[k]; k += 1
        ho_r = args[k]; k += 1
        yo_r = args[k] if have_wn else None

        cnt = c0_r[:, 0:1] + c1_r[:, 0:1]
        inv = 1.0 / jnp.maximum(cnt, 1.0)
        agg = (m0_r[...] + m1_r[...]) * inv
        if have_wr:
            agg = _dgT(agg, wr_r[...])
        hv = _dgT(h_r[...], wl_r[...]) + bl_r[...] + agg
        hv = jnp.maximum(hv, 0.0)
        ho_r[...] = hv
        if have_wn:
            yo_r[...] = _dgT(hv, wn_r[...])

    in_specs = [
        pl.BlockSpec((BN, din), lambda i: (i, 0)),
        pl.BlockSpec((BN, 128), lambda i: (i, 0)),
        pl.BlockSpec((BN, 128), lambda i: (i, 0)),
        pl.BlockSpec((BN, 128), lambda i: (i, 0)),
        pl.BlockSpec((BN, 128), lambda i: (i, 0)),
        pl.BlockSpec((dout, din), lambda i: (0, 0)),
        pl.BlockSpec((1, dout), lambda i: (0, 0)),
    ]
    args = [h, m0, m1, c0, c1, Wl, bl.reshape(1, -1)]
    if have_wr:
        in_specs.append(pl.BlockSpec((dout, 128), lambda i: (0, 0)))
        args.append(Wr)
    if have_wn:
        in_specs.append(pl.BlockSpec((128, dout), lambda i: (0, 0)))
        args.append(Wn)

    out_shape = [jax.ShapeDtypeStruct((N, dout), _F32)]
    out_specs = [pl.BlockSpec((BN, dout), lambda i: (i, 0))]
    if have_wn:
        out_shape.append(jax.ShapeDtypeStruct((N, 128), _F32))
        out_specs.append(pl.BlockSpec((BN, 128), lambda i: (i, 0)))

    res = pl.pallas_call(
        body,
        grid=(N // BN,),
        in_specs=in_specs,
        out_specs=out_specs,
        out_shape=out_shape,
    )(*args)
    return res


def _tc_pool_mlp(h4, batch2, l1W, l1b, l2W, l2b, l3W, l3b, l4W, l4b):
    nblk = N // BN

    def body(h_r, b_r, w1, b1, w2, b2, w3, b3, w4, b4, out_r, acc, cacc):
        i = pl.program_id(0)

        @pl.when(i == 0)
        def _():
            acc[...] = jnp.zeros((G, D), _F32)
            cacc[...] = jnp.zeros((G, D), _F32)

        mask = (b_r[...] == lax.broadcasted_iota(jnp.int32, (BN, G), 1)).astype(_F32)
        acc[...] += lax.dot_general(mask, h_r[...], (((0,), (0,)), ((), ())),
                                    preferred_element_type=_F32,
                                    precision=lax.Precision.HIGHEST)
        cacc[...] += lax.dot_general(mask, jnp.ones((BN, D), _F32),
                                     (((0,), (0,)), ((), ())),
                                     preferred_element_type=_F32,
                                     precision=lax.Precision.HIGHEST)

        @pl.when(i == nblk - 1)
        def _():
            g = acc[...] / jnp.maximum(cacc[...], 1.0)
            g = jnp.maximum(_dgT(g, w1[...]) + b1[...], 0.0)
            g = jnp.maximum(_dgT(g, w2[...]) + b2[...], 0.0)
            g = jnp.maximum(_dgT(g, w3[...]) + b3[...], 0.0)
            o = lax.dot_general(w4[...], g, (((1,), (1,)), ((), ())),
                                preferred_element_type=_F32,
                                precision=lax.Precision.HIGHEST)
            out_r[...] = o + b4[...]

    in_specs = [
        pl.BlockSpec((BN, D), lambda i: (i, 0)),
        pl.BlockSpec((BN, 1), lambda i: (i, 0)),
        pl.BlockSpec((128, 128), lambda i: (0, 0)),
        pl.BlockSpec((1, 128), lambda i: (0, 0)),
        pl.BlockSpec((64, 128), lambda i: (0, 0)),
        pl.BlockSpec((1, 64), lambda i: (0, 0)),
        pl.BlockSpec((64, 64), lambda i: (0, 0)),
        pl.BlockSpec((1, 64), lambda i: (0, 0)),
        pl.BlockSpec((1, 64), lambda i: (0, 0)),
        pl.BlockSpec((1, 1), lambda i: (0, 0)),
    ]
    out = pl.pallas_call(
        body,
        grid=(nblk,),
        in_specs=in_specs,
        out_specs=pl.BlockSpec((1, G), lambda i: (0, 0)),
        out_shape=jax.ShapeDtypeStruct((1, G), _F32),
        scratch_shapes=[pltpu.VMEM((G, D), _F32), pltpu.VMEM((G, D), _F32)],
    )(h4, batch2,
      l1W, l1b.reshape(1, -1), l2W, l2b.reshape(1, -1),
      l3W, l3b.reshape(1, -1), l4W, l4b.reshape(1, -1))
    return out


def kernel(x, edge_index, batch,
           conv1_Wl, conv1_bl, conv1_Wr,
           conv2_Wl, conv2_bl, conv2_Wr,
           conv3_Wl, conv3_bl, conv3_Wr,
           conv4_Wl, conv4_bl, conv4_Wr,
           lin1_W, lin1_b, lin2_W, lin2_b,
           lin3_W, lin3_b, lin4_W, lin4_b):
    pad = EPAD - E
    src3 = jnp.concatenate(
        [edge_index[0], jnp.zeros((pad,), jnp.int32)]).reshape(NW, NCH, C)
    dst3 = jnp.concatenate(
        [edge_index[1], jnp.full((pad,), N, jnp.int32)]).reshape(NW, NCH, C)
    zeros128 = jnp.zeros((C, D), _F32)
    ones128 = jnp.ones((C, D), _F32)

    cnt = _sc_segcnt(dst3, zeros128, ones128)
    c0 = cnt[:N]
    c1 = cnt[NP:NP + N]

    def seg(feat):
        m = _sc_segsum(feat, src3, dst3, zeros128)
        return m[:N], m[NP:NP + N]

    m0, m1 = seg(x)
    h1, y2 = _tc_layer(x, m0, m1, c0, c1, conv1_Wl, conv1_bl, conv1_Wr, conv2_Wr)
    m0, m1 = seg(y2)
    h2, y3 = _tc_layer(h1, m0, m1, c0, c1, conv2_Wl, conv2_bl, None, conv3_Wr)
    m0, m1 = seg(y3)
    h3, y4 = _tc_layer(h2, m0, m1, c0, c1, conv3_Wl, conv3_bl, None, conv4_Wr)
    m0, m1 = seg(y4)
    (h4,) = _tc_layer(h3, m0, m1, c0, c1, conv4_Wl, conv4_bl, None, None)

    out = _tc_pool_mlp(h4, batch.reshape(N, 1),
                       lin1_W, lin1_b, lin2_W, lin2_b,
                       lin3_W, lin3_b, lin4_W, lin4_b)
    return out.reshape(G)
